# baseline (speedup 1.0000x reference)
import functools

import jax
import jax.numpy as jnp
from jax import lax
from jax.experimental import pallas as pl
from jax.experimental.pallas import tpu as pltpu
from jax.experimental.pallas import tpu_sc as plsc

EPS = 1e-07
MIN_NORM = 1e-15
MAX_NORM = 1000000.0

_N = 10000
_D = 128
_E = 320000
_NC = 2
_NS = 16
_NW = _NC * _NS
_EPW = _E // _NW
_CH = 48
_NCHUNK = 209
_EPWP = _NCHUNK * _CH
_NP = _N + 8
_RPB = 624

_BN = 2000


def _cosh_sinh(t):
    tc = jnp.clip(t, -15.0, 15.0)
    e = jnp.exp(tc)
    em = jnp.exp(-tc)
    return 0.5 * (e + em), 0.5 * (e - em)


def _pre_body(x_ref, w_ref, y_ref, xt_ref):
    x = x_ref[...]
    W = w_ref[...]
    col = lax.broadcasted_iota(jnp.int32, x.shape, 1)
    is0 = col == 0
    x0 = x[:, 0:1]
    sumsq = jnp.sum(x * x, axis=1, keepdims=True)
    yn2 = sumsq - x0 * x0
    ynorm = jnp.maximum(jnp.sqrt(jnp.maximum(yn2, 0.0)), MIN_NORM)
    theta = jnp.maximum(x0, 1.0 + EPS)
    ach = jnp.log(theta + jnp.sqrt(theta * theta - 1.0))
    u = jnp.where(is0, 0.0, (ach / ynorm) * x)
    mu = lax.dot_general(u, W, (((1,), (1,)), ((), ())),
                         preferred_element_type=jnp.float32)
    mu1 = jnp.where(is0, 0.0, mu)
    xn2 = jnp.sum(mu1 * mu1, axis=1, keepdims=True)
    xnorm = jnp.maximum(jnp.sqrt(xn2), MIN_NORM)
    ch, sh = _cosh_sinh(xnorm)
    rest = (sh / xnorm) * mu1
    r2 = jnp.sum(rest * rest, axis=1, keepdims=True)
    firstp = jnp.sqrt(jnp.maximum(1.0 + r2, EPS))
    y_ref[...] = jnp.where(is0, firstp, rest)
    xt_ref[...] = jnp.where(is0, -x, x)


def _pre_call(x, W):
    grid = (_N // _BN,)
    return pl.pallas_call(
        _pre_body,
        grid=grid,
        in_specs=[
            pl.BlockSpec((_BN, _D), lambda i: (i, 0)),
            pl.BlockSpec((_D, _D), lambda i: (0, 0)),
        ],
        out_specs=[
            pl.BlockSpec((_BN, _D), lambda i: (i, 0)),
            pl.BlockSpec((_BN, _D), lambda i: (i, 0)),
        ],
        out_shape=[
            jax.ShapeDtypeStruct((_N, _D), jnp.float32),
            jax.ShapeDtypeStruct((_N, _D), jnp.float32),
        ],
    )(x, W)


def _post_body(a_ref, x_ref, h_ref):
    a = a_ref[0] + a_ref[1]
    x = x_ref[...]
    col = lax.broadcasted_iota(jnp.int32, x.shape, 1)
    is0 = col == 0
    a0 = a[:, 0:1]
    mu2 = jnp.sum(a * a, axis=1, keepdims=True) - 2.0 * a0 * a0
    normu = jnp.minimum(jnp.sqrt(jnp.maximum(mu2, EPS)), MAX_NORM)
    th = jnp.maximum(normu, MIN_NORM)
    ch, sh = _cosh_sinh(th)
    result = ch * x + (sh / th) * a
    r0 = result[:, 0:1]
    rsq = jnp.sum(result * result, axis=1, keepdims=True) - r0 * r0
    first = jnp.sqrt(jnp.maximum(1.0 + rsq, EPS))
    out = jnp.where(is0, first, result)
    p = jnp.maximum(jnp.where(is0, 0.0, out / (out[:, 0:1] + 1.0)), 0.0)
    sq = jnp.sum(p * p, axis=1, keepdims=True)
    h = jnp.where(is0, 1.0 + sq, 2.0 * p) / (1.0 - sq)
    h_ref[...] = jnp.maximum(h, 0.0)


def _post_call(aggr2, x):
    grid = (_N // _BN,)
    return pl.pallas_call(
        _post_body,
        grid=grid,
        in_specs=[
            pl.BlockSpec((_NC, _BN, _D), lambda i: (0, i, 0)),
            pl.BlockSpec((_BN, _D), lambda i: (i, 0)),
        ],
        out_specs=pl.BlockSpec((_BN, _D), lambda i: (i, 0)),
        out_shape=jax.ShapeDtypeStruct((_N, _D), jnp.float32),
    )(aggr2, x)


def _sc_rsqrt(v):
    i = plsc.bitcast(v, jnp.int32)
    r = plsc.bitcast(jnp.int32(0x5F3759DF) - (i >> 1), jnp.float32)
    for _ in range(3):
        r = r * (1.5 - 0.5 * v * r * r)
    return r


def _sc_sqrt(v):
    return v * _sc_rsqrt(v)


_LN2 = 0.6931471805599453
_SQRT2 = 1.4142135623730951


def _sc_log(t):
    bits = plsc.bitcast(t, jnp.int32)
    e = (bits >> 23) - 127
    m = plsc.bitcast((bits & jnp.int32(0x007FFFFF)) | jnp.int32(0x3F800000),
                     jnp.float32)
    big = m > _SQRT2
    m = jnp.where(big, 0.5 * m, m)
    ef = e.astype(jnp.float32) + jnp.where(big, 1.0, 0.0)
    z = (m - 1.0) / (m + 1.0)
    w = z * z
    p = 2.0 * z * (1.0 + w * (1.0 / 3.0 + w * (0.2 + w * (1.0 / 7.0 + w * (1.0 / 9.0)))))
    return ef * _LN2 + p


def _compute_chunk(xg, yg, msg, slot, slotv, lane, zero16, pvs, pvq):
    zidx = jnp.full((16,), 0, jnp.int32)

    def group_body(g):
        e0 = g * 16
        rows = e0 + lane
        gv = jnp.broadcast_to(g, (16,))

        for j in range(16):
            e = e0 + j
            sacc = [None] * 4
            qacc = [None] * 4
            for k in range(_D // 16):
                xk = xg[slot, e, pl.ds(k * 16, 16)]
                yk = yg[slot, e, pl.ds(k * 16, 16)]
                ps = xk * yk
                pq = xk * xk
                sacc[k % 4] = ps if sacc[k % 4] is None else sacc[k % 4] + ps
                qacc[k % 4] = pq if qacc[k % 4] is None else qacc[k % 4] + pq
            pvs[g, j, pl.ds(0, 16)] = (sacc[0] + sacc[1]) + (sacc[2] + sacc[3])
            pvq[g, j, pl.ds(0, 16)] = (qacc[0] + qacc[1]) + (qacc[2] + qacc[3])

        sa = [zero16] * 4
        qa = [zero16] * 4
        for d in range(16):
            cd = jnp.full((16,), d, jnp.int32)
            sa[d % 4] = sa[d % 4] + plsc.load_gather(pvs, [gv, lane, cd])
            qa[d % 4] = qa[d % 4] + plsc.load_gather(pvq, [gv, lane, cd])
        s = (sa[0] + sa[1]) + (sa[2] + sa[3])
        x0v = -plsc.load_gather(xg, [slotv, rows, zidx])
        y0v = plsc.load_gather(yg, [slotv, rows, zidx])
        q = (qa[0] + qa[1]) + (qa[2] + qa[3]) - 2.0 * x0v * x0v

        xy = jnp.minimum(s, -1.0 - EPS)
        theta = jnp.maximum(-s, 1.0 + EPS)
        rt = _sc_sqrt(theta * theta - 1.0)
        ach = _sc_log(theta + rt)
        dist = _sc_sqrt(jnp.minimum(ach * ach, 50.0))
        musq = q + 2.0 * xy * s - xy * xy
        normu = _sc_sqrt(jnp.maximum(musq, EPS))
        alpha = dist / normu
        beta = alpha * xy
        m0 = alpha * (s + y0v * x0v + xy * (y0v * y0v - 1.0)) / y0v

        for j in range(16):
            e = e0 + j
            a_s = alpha[j]
            b_s = beta[j]
            m_s = m0[j]
            for k in range(_D // 16):
                xk = xg[slot, e, pl.ds(k * 16, 16)]
                yk = yg[slot, e, pl.ds(k * 16, 16)]
                v = a_s * xk + b_s * yk
                if k == 0:
                    v = jnp.where(lane == 0, m_s, v)
                msg[e, pl.ds(k * 16, 16)] = v

    plsc.parallel_loop(0, _CH // 16, unroll=_CH // 16)(group_body)


def _sc_edge_body(xt_hbm, y_hbm, src3_hbm, dst3_hbm, out_hbm,
                  srcb, dstb, xg2, yg2, msg, pvs, pvq,
                  acc, semi, semj, semx, semy):
    core = lax.axis_index("c")
    sid = lax.axis_index("s")
    wid = core * _NS + sid
    lane = lax.iota(jnp.int32, 16)
    zero16 = jnp.zeros((16,), jnp.float32)

    def zfill(r, carry):
        for k in range(_D // 16):
            msg[r, pl.ds(k * 16, 16)] = zero16
        return carry
    lax.fori_loop(0, _CH, zfill, 0, unroll=False)
    for b in range(_RPB // _CH):
        pltpu.sync_copy(msg, acc.at[pl.ds(sid * _RPB + b * _CH, _CH)])

    @pl.when(sid == _NS - 1)
    def _zero_tail():
        pltpu.sync_copy(msg.at[pl.ds(0, 24)], acc.at[pl.ds(_NS * _RPB, 24)])

    plsc.subcore_barrier()

    pltpu.async_copy(src3_hbm.at[wid, 0], srcb.at[0], semi.at[0])
    pltpu.async_copy(dst3_hbm.at[wid, 0], dstb.at[0], semj.at[0])

    def tloop(t, carry):
        @pl.when(t + 1 < _NCHUNK)
        def _prefetch_idx():
            s3 = (t + 1) % 3
            pltpu.async_copy(src3_hbm.at[wid, t + 1], srcb.at[s3],
                             semi.at[s3])
            pltpu.async_copy(dst3_hbm.at[wid, t + 1], dstb.at[s3],
                             semj.at[s3])

        @pl.when(t < _NCHUNK)
        def _issue_gather():
            s3 = t % 3
            slot = t & 1
            pltpu.make_async_copy(src3_hbm.at[wid, t], srcb.at[s3],
                                  semi.at[s3]).wait()
            pltpu.make_async_copy(dst3_hbm.at[wid, t], dstb.at[s3],
                                  semj.at[s3]).wait()
            pltpu.async_copy(xt_hbm.at[dstb.at[s3]], xg2.at[slot],
                             semx.at[slot])
            pltpu.async_copy(y_hbm.at[srcb.at[s3]], yg2.at[slot],
                             semy.at[slot])

        @pl.when(t >= 1)
        def _consume():
            c = t - 1
            s3 = c % 3
            slot = c & 1
            pltpu.make_async_copy(xt_hbm.at[dstb.at[s3]], xg2.at[slot],
                                  semx.at[slot]).wait()
            pltpu.make_async_copy(y_hbm.at[srcb.at[s3]], yg2.at[slot],
                                  semy.at[slot]).wait()
            slotv = jnp.broadcast_to(slot, (16,))
            _compute_chunk(xg2, yg2, msg, slot, slotv, lane, zero16,
                           pvs, pvq)
            pltpu.sync_copy(msg, acc.at[dstb.at[s3]], add=True)

        return carry

    lax.fori_loop(0, _NCHUNK + 1, tloop, 0, unroll=False)

    plsc.subcore_barrier()
    pltpu.sync_copy(acc.at[pl.ds(sid * _RPB, _RPB)],
                    out_hbm.at[core, pl.ds(sid * _RPB, _RPB)])

    @pl.when(sid == _NS - 1)
    def _dump_tail():
        pltpu.sync_copy(acc.at[pl.ds(_NS * _RPB, 16)],
                        out_hbm.at[core, pl.ds(_NS * _RPB, 16)])


@functools.partial(
    pl.kernel,
    out_type=jax.ShapeDtypeStruct((_NC, _N, _D), jnp.float32),
    mesh=plsc.VectorSubcoreMesh(core_axis_name="c", subcore_axis_name="s",
                                num_cores=_NC, num_subcores=_NS),
    compiler_params=pltpu.CompilerParams(needs_layout_passes=False),
    scratch_types=[
        pltpu.VMEM((3, _CH), jnp.int32),
        pltpu.VMEM((3, _CH), jnp.int32),
        pltpu.VMEM((2, _CH, _D), jnp.float32),
        pltpu.VMEM((2, _CH, _D), jnp.float32),
        pltpu.VMEM((_CH, _D), jnp.float32),
        pltpu.VMEM((_CH // 16, 16, 16), jnp.float32),
        pltpu.VMEM((_CH // 16, 16, 16), jnp.float32),
        pltpu.VMEM_SHARED((_NP, _D), jnp.float32),
        pltpu.SemaphoreType.DMA((3,)),
        pltpu.SemaphoreType.DMA((3,)),
        pltpu.SemaphoreType.DMA((2,)),
        pltpu.SemaphoreType.DMA((2,)),
    ],
)
def _sc_edge(xt_hbm, y_hbm, src3_hbm, dst3_hbm, out_hbm,
             srcb, dstb, xg2, yg2, msg, pvs, pvq,
             acc, semi, semj, semx, semy):
    _sc_edge_body(xt_hbm, y_hbm, src3_hbm, dst3_hbm, out_hbm,
                  srcb, dstb, xg2, yg2, msg, pvs, pvq,
                  acc, semi, semj, semx, semy)


def kernel(x, edge_index, W1, W2):
    pad = jnp.full((_NW, _EPWP - _EPW), _N, jnp.int32)
    src3 = jnp.concatenate(
        [edge_index[0].astype(jnp.int32).reshape(_NW, _EPW), pad], axis=1
    ).reshape(_NW, _NCHUNK, _CH)
    dst3 = jnp.concatenate(
        [edge_index[1].astype(jnp.int32).reshape(_NW, _EPW), pad], axis=1
    ).reshape(_NW, _NCHUNK, _CH)
    zrows = jnp.zeros((_NP - _N, _D), jnp.float32)
    h = x
    for W in (W1, W2):
        Y, XT = _pre_call(h, W)
        Yp = jnp.concatenate([Y, zrows], axis=0)
        XTp = jnp.concatenate([XT, zrows], axis=0)
        aggr2 = _sc_edge(XTp, Yp, src3, dst3)
        h = _post_call(aggr2, h)
    return h


# speedup vs baseline: 1.4805x; 1.4805x over previous
import functools

import jax
import jax.numpy as jnp
from jax import lax
from jax.experimental import pallas as pl
from jax.experimental.pallas import tpu as pltpu
from jax.experimental.pallas import tpu_sc as plsc

EPS = 1e-07
MIN_NORM = 1e-15
MAX_NORM = 1000000.0

_N = 10000
_D = 128
_E = 320000
_NC = 2
_NS = 16
_NW = _NC * _NS
_EPW = _E // _NW
_CH = 48
_NCHUNK = 209
_EPWP = _NCHUNK * _CH
_NP = _N + 8
_RPB = 624

_BN = 2000


def _cosh_sinh(t):
    tc = jnp.clip(t, -15.0, 15.0)
    e = jnp.exp(tc)
    em = jnp.exp(-tc)
    return 0.5 * (e + em), 0.5 * (e - em)


def _pre_body(x_ref, w_ref, y_ref, xt_ref):
    x = x_ref[...]
    W = w_ref[...]
    col = lax.broadcasted_iota(jnp.int32, x.shape, 1)
    is0 = col == 0
    x0 = x[:, 0:1]
    sumsq = jnp.sum(x * x, axis=1, keepdims=True)
    yn2 = sumsq - x0 * x0
    ynorm = jnp.maximum(jnp.sqrt(jnp.maximum(yn2, 0.0)), MIN_NORM)
    theta = jnp.maximum(x0, 1.0 + EPS)
    ach = jnp.log(theta + jnp.sqrt(theta * theta - 1.0))
    u = jnp.where(is0, 0.0, (ach / ynorm) * x)
    mu = lax.dot_general(u, W, (((1,), (1,)), ((), ())),
                         preferred_element_type=jnp.float32)
    mu1 = jnp.where(is0, 0.0, mu)
    xn2 = jnp.sum(mu1 * mu1, axis=1, keepdims=True)
    xnorm = jnp.maximum(jnp.sqrt(xn2), MIN_NORM)
    ch, sh = _cosh_sinh(xnorm)
    rest = (sh / xnorm) * mu1
    r2 = jnp.sum(rest * rest, axis=1, keepdims=True)
    firstp = jnp.sqrt(jnp.maximum(1.0 + r2, EPS))
    y_ref[...] = jnp.where(is0, firstp, rest)
    xt_ref[...] = jnp.where(is0, -x, x)


def _pre_call(x, W):
    grid = (_N // _BN,)
    return pl.pallas_call(
        _pre_body,
        grid=grid,
        in_specs=[
            pl.BlockSpec((_BN, _D), lambda i: (i, 0)),
            pl.BlockSpec((_D, _D), lambda i: (0, 0)),
        ],
        out_specs=[
            pl.BlockSpec((_BN, _D), lambda i: (i, 0)),
            pl.BlockSpec((_BN, _D), lambda i: (i, 0)),
        ],
        out_shape=[
            jax.ShapeDtypeStruct((_N, _D), jnp.float32),
            jax.ShapeDtypeStruct((_N, _D), jnp.float32),
        ],
    )(x, W)


def _post_body(a_ref, x_ref, h_ref):
    a = a_ref[0] + a_ref[1]
    x = x_ref[...]
    col = lax.broadcasted_iota(jnp.int32, x.shape, 1)
    is0 = col == 0
    a0 = a[:, 0:1]
    mu2 = jnp.sum(a * a, axis=1, keepdims=True) - 2.0 * a0 * a0
    normu = jnp.minimum(jnp.sqrt(jnp.maximum(mu2, EPS)), MAX_NORM)
    th = jnp.maximum(normu, MIN_NORM)
    ch, sh = _cosh_sinh(th)
    result = ch * x + (sh / th) * a
    r0 = result[:, 0:1]
    rsq = jnp.sum(result * result, axis=1, keepdims=True) - r0 * r0
    first = jnp.sqrt(jnp.maximum(1.0 + rsq, EPS))
    out = jnp.where(is0, first, result)
    p = jnp.maximum(jnp.where(is0, 0.0, out / (out[:, 0:1] + 1.0)), 0.0)
    sq = jnp.sum(p * p, axis=1, keepdims=True)
    h = jnp.where(is0, 1.0 + sq, 2.0 * p) / (1.0 - sq)
    h_ref[...] = jnp.maximum(h, 0.0)


def _post_call(aggr2, x):
    grid = (_N // _BN,)
    return pl.pallas_call(
        _post_body,
        grid=grid,
        in_specs=[
            pl.BlockSpec((_NC, _BN, _D), lambda i: (0, i, 0)),
            pl.BlockSpec((_BN, _D), lambda i: (i, 0)),
        ],
        out_specs=pl.BlockSpec((_BN, _D), lambda i: (i, 0)),
        out_shape=jax.ShapeDtypeStruct((_N, _D), jnp.float32),
    )(aggr2, x)


def _sc_rsqrt(v):
    i = plsc.bitcast(v, jnp.int32)
    r = plsc.bitcast(jnp.int32(0x5F3759DF) - (i >> 1), jnp.float32)
    for _ in range(3):
        r = r * (1.5 - 0.5 * v * r * r)
    return r


def _sc_sqrt(v):
    return v * _sc_rsqrt(v)


_LN2 = 0.6931471805599453
_SQRT2 = 1.4142135623730951


def _sc_log(t):
    bits = plsc.bitcast(t, jnp.int32)
    e = (bits >> 23) - 127
    m = plsc.bitcast((bits & jnp.int32(0x007FFFFF)) | jnp.int32(0x3F800000),
                     jnp.float32)
    big = m > _SQRT2
    m = jnp.where(big, 0.5 * m, m)
    ef = e.astype(jnp.float32) + jnp.where(big, 1.0, 0.0)
    z = (m - 1.0) / (m + 1.0)
    w = z * z
    p = 2.0 * z * (1.0 + w * (1.0 / 3.0 + w * (0.2 + w * (1.0 / 7.0 + w * (1.0 / 9.0)))))
    return ef * _LN2 + p


def _compute_chunk(xg, yg, msg, slot, slotv, lane, zero16, pvs, pvq):
    zidx = jnp.full((16,), 0, jnp.int32)

    def group_body(g):
        e0 = g * 16
        rows = e0 + lane
        gv = jnp.broadcast_to(g, (16,))

        for j in range(16):
            e = e0 + j
            sacc = [None] * 4
            qacc = [None] * 4
            for k in range(_D // 16):
                xk = xg[slot, e, pl.ds(k * 16, 16)]
                yk = yg[slot, e, pl.ds(k * 16, 16)]
                ps = xk * yk
                pq = xk * xk
                sacc[k % 4] = ps if sacc[k % 4] is None else sacc[k % 4] + ps
                qacc[k % 4] = pq if qacc[k % 4] is None else qacc[k % 4] + pq
            pvs[g, j, pl.ds(0, 16)] = (sacc[0] + sacc[1]) + (sacc[2] + sacc[3])
            pvq[g, j, pl.ds(0, 16)] = (qacc[0] + qacc[1]) + (qacc[2] + qacc[3])

        sa = [zero16] * 4
        qa = [zero16] * 4
        for d in range(16):
            cd = jnp.full((16,), d, jnp.int32)
            sa[d % 4] = sa[d % 4] + plsc.load_gather(pvs, [gv, lane, cd])
            qa[d % 4] = qa[d % 4] + plsc.load_gather(pvq, [gv, lane, cd])
        s = (sa[0] + sa[1]) + (sa[2] + sa[3])
        x0v = -plsc.load_gather(xg, [slotv, rows, zidx])
        y0v = plsc.load_gather(yg, [slotv, rows, zidx])
        q = (qa[0] + qa[1]) + (qa[2] + qa[3]) - 2.0 * x0v * x0v

        xy = jnp.minimum(s, -1.0 - EPS)
        theta = jnp.maximum(-s, 1.0 + EPS)
        rt = _sc_sqrt(theta * theta - 1.0)
        ach = _sc_log(theta + rt)
        dist = _sc_sqrt(jnp.minimum(ach * ach, 50.0))
        musq = q + 2.0 * xy * s - xy * xy
        normu = _sc_sqrt(jnp.maximum(musq, EPS))
        alpha = dist / normu
        beta = alpha * xy
        m0 = alpha * (s + y0v * x0v + xy * (y0v * y0v - 1.0)) / y0v

        for j in range(16):
            e = e0 + j
            a_s = alpha[j]
            b_s = beta[j]
            m_s = m0[j]
            for k in range(_D // 16):
                xk = xg[slot, e, pl.ds(k * 16, 16)]
                yk = yg[slot, e, pl.ds(k * 16, 16)]
                v = a_s * xk + b_s * yk
                if k == 0:
                    v = jnp.where(lane == 0, m_s, v)
                msg[e, pl.ds(k * 16, 16)] = v

    plsc.parallel_loop(0, _CH // 16)(group_body)


def _sc_edge_body(xt_hbm, y_hbm, src3_hbm, dst3_hbm, out_hbm,
                  srcb, dstb, xg2, yg2, msg, pvs, pvq,
                  acc, semi, semj, semx, semy):
    core = lax.axis_index("c")
    sid = lax.axis_index("s")
    wid = core * _NS + sid
    lane = lax.iota(jnp.int32, 16)
    zero16 = jnp.zeros((16,), jnp.float32)

    def zfill(r, carry):
        for k in range(_D // 16):
            msg[r, pl.ds(k * 16, 16)] = zero16
        return carry
    lax.fori_loop(0, _CH, zfill, 0, unroll=False)
    for b in range(_RPB // _CH):
        pltpu.sync_copy(msg, acc.at[pl.ds(sid * _RPB + b * _CH, _CH)])

    @pl.when(sid == _NS - 1)
    def _zero_tail():
        pltpu.sync_copy(msg.at[pl.ds(0, 24)], acc.at[pl.ds(_NS * _RPB, 24)])

    plsc.subcore_barrier()

    pltpu.async_copy(src3_hbm.at[wid, 0], srcb.at[0], semi.at[0])
    pltpu.async_copy(dst3_hbm.at[wid, 0], dstb.at[0], semj.at[0])

    def tloop(t, carry):
        @pl.when(t + 1 < _NCHUNK)
        def _prefetch_idx():
            s3 = (t + 1) % 3
            pltpu.async_copy(src3_hbm.at[wid, t + 1], srcb.at[s3],
                             semi.at[s3])
            pltpu.async_copy(dst3_hbm.at[wid, t + 1], dstb.at[s3],
                             semj.at[s3])

        @pl.when(t < _NCHUNK)
        def _issue_gather():
            s3 = t % 3
            slot = t & 1
            pltpu.make_async_copy(src3_hbm.at[wid, t], srcb.at[s3],
                                  semi.at[s3]).wait()
            pltpu.make_async_copy(dst3_hbm.at[wid, t], dstb.at[s3],
                                  semj.at[s3]).wait()
            pltpu.async_copy(xt_hbm.at[dstb.at[s3]], xg2.at[slot],
                             semx.at[slot])
            pltpu.async_copy(y_hbm.at[srcb.at[s3]], yg2.at[slot],
                             semy.at[slot])

        @pl.when(t >= 1)
        def _consume():
            c = t - 1
            s3 = c % 3
            slot = c & 1
            pltpu.make_async_copy(xt_hbm.at[dstb.at[s3]], xg2.at[slot],
                                  semx.at[slot]).wait()
            pltpu.make_async_copy(y_hbm.at[srcb.at[s3]], yg2.at[slot],
                                  semy.at[slot]).wait()
            slotv = jnp.broadcast_to(slot, (16,))
            _compute_chunk(xg2, yg2, msg, slot, slotv, lane, zero16,
                           pvs, pvq)
            pltpu.sync_copy(msg, acc.at[dstb.at[s3]], add=True)

        return carry

    lax.fori_loop(0, _NCHUNK + 1, tloop, 0, unroll=False)

    plsc.subcore_barrier()
    pltpu.sync_copy(acc.at[pl.ds(sid * _RPB, _RPB)],
                    out_hbm.at[core, pl.ds(sid * _RPB, _RPB)])

    @pl.when(sid == _NS - 1)
    def _dump_tail():
        pltpu.sync_copy(acc.at[pl.ds(_NS * _RPB, 16)],
                        out_hbm.at[core, pl.ds(_NS * _RPB, 16)])


@functools.partial(
    pl.kernel,
    out_type=jax.ShapeDtypeStruct((_NC, _N, _D), jnp.float32),
    mesh=plsc.VectorSubcoreMesh(core_axis_name="c", subcore_axis_name="s",
                                num_cores=_NC, num_subcores=_NS),
    compiler_params=pltpu.CompilerParams(needs_layout_passes=False),
    scratch_types=[
        pltpu.VMEM((3, _CH), jnp.int32),
        pltpu.VMEM((3, _CH), jnp.int32),
        pltpu.VMEM((2, _CH, _D), jnp.float32),
        pltpu.VMEM((2, _CH, _D), jnp.float32),
        pltpu.VMEM((_CH, _D), jnp.float32),
        pltpu.VMEM((_CH // 16, 16, 16), jnp.float32),
        pltpu.VMEM((_CH // 16, 16, 16), jnp.float32),
        pltpu.VMEM_SHARED((_NP, _D), jnp.float32),
        pltpu.SemaphoreType.DMA((3,)),
        pltpu.SemaphoreType.DMA((3,)),
        pltpu.SemaphoreType.DMA((2,)),
        pltpu.SemaphoreType.DMA((2,)),
    ],
)
def _sc_edge(xt_hbm, y_hbm, src3_hbm, dst3_hbm, out_hbm,
             srcb, dstb, xg2, yg2, msg, pvs, pvq,
             acc, semi, semj, semx, semy):
    _sc_edge_body(xt_hbm, y_hbm, src3_hbm, dst3_hbm, out_hbm,
                  srcb, dstb, xg2, yg2, msg, pvs, pvq,
                  acc, semi, semj, semx, semy)


def kernel(x, edge_index, W1, W2):
    pad = jnp.full((_NW, _EPWP - _EPW), _N, jnp.int32)
    src3 = jnp.concatenate(
        [edge_index[0].astype(jnp.int32).reshape(_NW, _EPW), pad], axis=1
    ).reshape(_NW, _NCHUNK, _CH)
    dst3 = jnp.concatenate(
        [edge_index[1].astype(jnp.int32).reshape(_NW, _EPW), pad], axis=1
    ).reshape(_NW, _NCHUNK, _CH)
    zrows = jnp.zeros((_NP - _N, _D), jnp.float32)
    h = x
    for W in (W1, W2):
        Y, XT = _pre_call(h, W)
        Yp = jnp.concatenate([Y, zrows], axis=0)
        XTp = jnp.concatenate([XT, zrows], axis=0)
        aggr2 = _sc_edge(XTp, Yp, src3, dst3)
        h = _post_call(aggr2, h)
    return h
